# baseline (speedup 1.0000x reference)
import functools

import jax
import jax.numpy as jnp
from jax import lax
from jax.experimental import pallas as pl
from jax.experimental.pallas import tpu as pltpu
from jax.experimental.pallas import tpu_sc as plsc

NW = 32
CHUNK = 448
BN = 1024
PW = 2048
S = 2
NPAD = 100352


def _sc_gather(table, idx):
    total = idx.shape[0]
    per_w = total // NW
    iters = per_w // CHUNK
    assert iters % 2 == 0 and iters * CHUNK == per_w
    nc = 2
    dt = table.dtype

    mesh = plsc.VectorSubcoreMesh(core_axis_name="c", subcore_axis_name="s")

    @functools.partial(
        pl.kernel, mesh=mesh,
        out_type=jax.ShapeDtypeStruct((total, 128), dt),
        scratch_types=[
            pltpu.VMEM((per_w,), jnp.int32),
            pltpu.VMEM((CHUNK, 128), dt),
            pltpu.VMEM((CHUNK, 128), dt),
            pltpu.SemaphoreType.DMA,
            pltpu.SemaphoreType.DMA,
        ],
    )
    def gather_k(idx_hbm, tab_hbm, out_hbm, idx_all, buf0, buf1, sem0, sem1):
        wid = lax.axis_index("s") * nc + lax.axis_index("c")
        base = wid * per_w
        pltpu.sync_copy(idx_hbm.at[pl.ds(base, per_w)], idx_all)

        bufs = (buf0, buf1)
        sems = (sem0, sem1)

        def gcopy(j, b):
            return pltpu.make_async_copy(
                tab_hbm.at[idx_all.at[pl.ds(j * CHUNK, CHUNK)]], bufs[b], sems[b])

        def step(j, b):
            @pl.when(j + 1 < iters)
            def _():
                gcopy(j + 1, 1 - b).start()
            gcopy(j, b).wait()
            pltpu.sync_copy(bufs[b], out_hbm.at[pl.ds(base + j * CHUNK, CHUNK)])

        gcopy(0, 0).start()

        def pair(i2, carry):
            step(i2 * 2, 0)
            step(i2 * 2 + 1, 1)
            return carry

        lax.fori_loop(0, iters // 2, pair, 0)

    return gather_k(idx, table)


def _unpack_bf16_pair(words):
    u = lax.bitcast_convert_type(words, jnp.uint32)
    hi = lax.bitcast_convert_type(u & jnp.uint32(0xFFFF0000), jnp.float32)
    lo = lax.bitcast_convert_type(u << 16, jnp.float32)
    return hi, lo


def kernel(xyz, h, x, knn_indices,
           Wz_pos, bz_pos, Wz_lin, bz_lin,
           Wr_pos, br_pos, Wr_lin, br_lin,
           Wq_pos, bq_pos, Wq_lin, bq_lin):
    h = h.astype(jnp.float32)
    x = x.astype(jnp.float32)
    B, N, _ = xyz.shape
    k = knn_indices.shape[2]
    H = h.shape[1]
    CIN = 2 * H
    Npad = NPAD
    NC = Npad // S
    grid = NC // BN

    def pack_body(h_ref, x_ref, xyz_ref, tab_ref):
        hb = lax.bitcast_convert_type(
            h_ref[...].astype(jnp.bfloat16), jnp.uint16).astype(jnp.uint32)
        xb = lax.bitcast_convert_type(
            x_ref[...].astype(jnp.bfloat16), jnp.uint16).astype(jnp.uint32)
        packed = lax.bitcast_convert_type((hb << 16) | xb, jnp.float32)
        packed_t = jnp.swapaxes(packed, 0, 1)
        xyz16 = jnp.pad(xyz_ref[...], ((0, 0), (0, 13)))
        zer = jnp.zeros((PW, 128 - H - 16), jnp.float32)
        tab_ref[...] = jnp.concatenate([packed_t, xyz16, zer], axis=1)

    xyz0 = xyz[0]
    h0 = h[0]
    x0 = x[0]
    tab1 = pl.pallas_call(
        pack_body,
        grid=(Npad // PW,),
        in_specs=[
            pl.BlockSpec((H, PW), lambda i: (0, i)),
            pl.BlockSpec((H, PW), lambda i: (0, i)),
            pl.BlockSpec((PW, 3), lambda i: (i, 0)),
        ],
        out_specs=pl.BlockSpec((PW, 128), lambda i: (i, 0)),
        out_shape=jax.ShapeDtypeStruct((Npad, 128), jnp.float32),
    )(h0, x0, xyz0)

    idx_km = jnp.pad(knn_indices[0].astype(jnp.int32).T,
                     ((0, 0), (0, Npad - N)))
    idx_chunks = [idx_km[:, c * NC:(c + 1) * NC].reshape(k * NC)
                  for c in range(S)]

    def padpos(W, rows):
        return jnp.pad(W.astype(jnp.float32), ((0, rows - 3), (0, 0)))
    bz2, br2, bq2 = (b.astype(jnp.float32).reshape(1, CIN)
                     for b in (bz_pos, br_pos, bq_pos))
    bzl2, brl2, bql2 = (b.astype(jnp.float32).reshape(1, H)
                        for b in (bz_lin, br_lin, bq_lin))
    full = lambda shape: pl.BlockSpec(shape, lambda i: (0, 0))

    def zr_body(nbr_ref, ctr_ref,
                wzp_ref, bz_ref, wzl_ref, bzl_ref,
                wrp_ref, br_ref, wrl_ref, brl_ref,
                z_ref, tab2_ref, rel_ref):
        nb = nbr_ref[...]
        ctr = ctr_ref[...]
        hc, xc_feat = _unpack_bf16_pair(ctr[:, :H])
        xc = ctr[:, H:H + 16]
        feats = []
        for j in range(k):
            fh, fl = _unpack_bf16_pair(nb[j, :, :H])
            feats.append(jnp.concatenate([fh, fl], axis=1))
        relf = jnp.concatenate([nb[j, :, H:H + 16] - xc for j in range(k)],
                               axis=0)
        wz = jnp.tanh(jnp.dot(relf, wzp_ref[...],
                              preferred_element_type=jnp.float32) + bz_ref[...])
        aggz = sum(wz[j * BN:(j + 1) * BN] * feats[j] for j in range(k))
        zz = jax.nn.sigmoid(jnp.dot(aggz, wzl_ref[...],
                                    preferred_element_type=jnp.float32) + bzl_ref[...])
        wr = jnp.tanh(jnp.dot(relf, wrp_ref[...],
                              preferred_element_type=jnp.float32) + br_ref[...])
        aggr = sum(wr[j * BN:(j + 1) * BN] * feats[j] for j in range(k))
        rr = jax.nn.sigmoid(jnp.dot(aggr, wrl_ref[...],
                                    preferred_element_type=jnp.float32) + brl_ref[...])
        z_ref[...] = zz
        tab2_ref[...] = jnp.concatenate([rr * hc, xc_feat], axis=1)
        rel_ref[...] = relf[:, :8].reshape(k, BN, 8)

    def phase2(nbr1_c, ctr_c):
        return pl.pallas_call(
            zr_body,
            grid=(grid,),
            in_specs=[
                pl.BlockSpec((k, BN, 128), lambda i: (0, i, 0)),
                pl.BlockSpec((BN, 128), lambda i: (i, 0)),
                full((16, CIN)), full((1, CIN)), full((CIN, H)), full((1, H)),
                full((16, CIN)), full((1, CIN)), full((CIN, H)), full((1, H)),
            ],
            out_specs=[
                pl.BlockSpec((BN, H), lambda i: (i, 0)),
                pl.BlockSpec((BN, CIN), lambda i: (i, 0)),
                pl.BlockSpec((k, BN, 8), lambda i: (0, i, 0)),
            ],
            out_shape=[
                jax.ShapeDtypeStruct((NC, H), jnp.float32),
                jax.ShapeDtypeStruct((NC, CIN), jnp.float32),
                jax.ShapeDtypeStruct((k, NC, 8), jnp.float32),
            ],
        )(nbr1_c, ctr_c,
          padpos(Wz_pos, 16), bz2, Wz_lin.astype(jnp.float32), bzl2,
          padpos(Wr_pos, 16), br2, Wr_lin.astype(jnp.float32), brl2)

    def q_body(nbr_ref, rel_ref, z_ref, ctr_ref,
               wqp_ref, bq_ref, wql_ref, bql_ref, out_ref):
        nb = nbr_ref[...]
        hc, _ = _unpack_bf16_pair(ctr_ref[...][:, :H])
        wq = jnp.tanh(jnp.dot(rel_ref[...].reshape(k * BN, 8), wqp_ref[...],
                              preferred_element_type=jnp.float32) + bq_ref[...])
        aggq = sum(wq[j * BN:(j + 1) * BN] * nb[j] for j in range(k))
        qq = jnp.tanh(jnp.dot(aggq, wql_ref[...],
                              preferred_element_type=jnp.float32) + bql_ref[...])
        zz = z_ref[...]
        hn = (1.0 - zz) * hc + zz * qq
        eye = jnp.eye(H, dtype=jnp.float32)
        out_ref[...] = lax.dot_general(eye, hn, (((1,), (1,)), ((), ())),
                                       preferred_element_type=jnp.float32)

    def phase4(nbr2_c, rel8_c, z_c, ctr_c):
        return pl.pallas_call(
            q_body,
            grid=(grid,),
            in_specs=[
                pl.BlockSpec((k, BN, CIN), lambda i: (0, i, 0)),
                pl.BlockSpec((k, BN, 8), lambda i: (0, i, 0)),
                pl.BlockSpec((BN, H), lambda i: (i, 0)),
                pl.BlockSpec((BN, 128), lambda i: (i, 0)),
                full((8, CIN)), full((1, CIN)), full((CIN, H)), full((1, H)),
            ],
            out_specs=pl.BlockSpec((H, BN), lambda i: (0, i)),
            out_shape=jax.ShapeDtypeStruct((H, NC), jnp.float32),
        )(nbr2_c, rel8_c, z_c, ctr_c,
          padpos(Wq_pos, 8), bq2, Wq_lin.astype(jnp.float32), bql2)

    def csl(a, c):
        return a[c * NC:(c + 1) * NC]

    nbr1 = [_sc_gather(tab1, idx_chunks[c]).reshape(k, NC, 128)
            for c in range(S)]
    p2 = [phase2(nbr1[c], csl(tab1, c)) for c in range(S)]
    tab2 = jnp.concatenate([p[1] for p in p2], axis=0)

    nbr2 = [_sc_gather(tab2, idx_chunks[c]).reshape(k, NC, CIN)
            for c in range(S)]
    outs = [phase4(nbr2[c], p2[c][2], p2[c][0], csl(tab1, c))
            for c in range(S)]

    h_new = jnp.concatenate(outs, axis=1)
    return h_new[:, :N][None]


# speedup vs baseline: 1.0293x; 1.0293x over previous
import functools

import jax
import jax.numpy as jnp
from jax import lax
from jax.experimental import pallas as pl
from jax.experimental.pallas import tpu as pltpu
from jax.experimental.pallas import tpu_sc as plsc

NW = 32
CHUNK = 448
BN = 1792
PW = 2048
S = 2
NPAD = 100352


def _sc_gather(table, idx):
    total = idx.shape[0]
    per_w = total // NW
    iters = per_w // CHUNK
    assert iters % 2 == 0 and iters * CHUNK == per_w
    nc = 2
    dt = table.dtype

    mesh = plsc.VectorSubcoreMesh(core_axis_name="c", subcore_axis_name="s")

    @functools.partial(
        pl.kernel, mesh=mesh,
        out_type=jax.ShapeDtypeStruct((total, 128), dt),
        scratch_types=[
            pltpu.VMEM((per_w,), jnp.int32),
            pltpu.VMEM((CHUNK, 128), dt),
            pltpu.VMEM((CHUNK, 128), dt),
            pltpu.SemaphoreType.DMA,
            pltpu.SemaphoreType.DMA,
        ],
    )
    def gather_k(idx_hbm, tab_hbm, out_hbm, idx_all, buf0, buf1, sem0, sem1):
        wid = lax.axis_index("s") * nc + lax.axis_index("c")
        base = wid * per_w
        pltpu.sync_copy(idx_hbm.at[pl.ds(base, per_w)], idx_all)

        bufs = (buf0, buf1)
        sems = (sem0, sem1)

        def gcopy(j, b):
            return pltpu.make_async_copy(
                tab_hbm.at[idx_all.at[pl.ds(j * CHUNK, CHUNK)]], bufs[b], sems[b])

        def step(j, b):
            @pl.when(j + 1 < iters)
            def _():
                gcopy(j + 1, 1 - b).start()
            gcopy(j, b).wait()
            pltpu.sync_copy(bufs[b], out_hbm.at[pl.ds(base + j * CHUNK, CHUNK)])

        gcopy(0, 0).start()

        def pair(i2, carry):
            step(i2 * 2, 0)
            step(i2 * 2 + 1, 1)
            return carry

        lax.fori_loop(0, iters // 2, pair, 0)

    return gather_k(idx, table)


def _unpack_bf16_pair(words):
    u = lax.bitcast_convert_type(words, jnp.uint32)
    hi = lax.bitcast_convert_type(u & jnp.uint32(0xFFFF0000), jnp.float32)
    lo = lax.bitcast_convert_type(u << 16, jnp.float32)
    return hi, lo


def kernel(xyz, h, x, knn_indices,
           Wz_pos, bz_pos, Wz_lin, bz_lin,
           Wr_pos, br_pos, Wr_lin, br_lin,
           Wq_pos, bq_pos, Wq_lin, bq_lin):
    h = h.astype(jnp.float32)
    x = x.astype(jnp.float32)
    B, N, _ = xyz.shape
    k = knn_indices.shape[2]
    H = h.shape[1]
    CIN = 2 * H
    Npad = NPAD
    NC = Npad // S
    grid = NC // BN

    def pack_body(h_ref, x_ref, xyz_ref, tab_ref):
        hb = lax.bitcast_convert_type(
            h_ref[...].astype(jnp.bfloat16), jnp.uint16).astype(jnp.uint32)
        xb = lax.bitcast_convert_type(
            x_ref[...].astype(jnp.bfloat16), jnp.uint16).astype(jnp.uint32)
        packed = lax.bitcast_convert_type((hb << 16) | xb, jnp.float32)
        packed_t = jnp.swapaxes(packed, 0, 1)
        xyz16 = jnp.pad(xyz_ref[...], ((0, 0), (0, 13)))
        zer = jnp.zeros((PW, 128 - H - 16), jnp.float32)
        tab_ref[...] = jnp.concatenate([packed_t, xyz16, zer], axis=1)

    xyz0 = xyz[0]
    h0 = h[0]
    x0 = x[0]
    tab1 = pl.pallas_call(
        pack_body,
        grid=(Npad // PW,),
        in_specs=[
            pl.BlockSpec((H, PW), lambda i: (0, i)),
            pl.BlockSpec((H, PW), lambda i: (0, i)),
            pl.BlockSpec((PW, 3), lambda i: (i, 0)),
        ],
        out_specs=pl.BlockSpec((PW, 128), lambda i: (i, 0)),
        out_shape=jax.ShapeDtypeStruct((Npad, 128), jnp.float32),
    )(h0, x0, xyz0)

    idx_km = jnp.pad(knn_indices[0].astype(jnp.int32).T,
                     ((0, 0), (0, Npad - N)))
    idx_chunks = [idx_km[:, c * NC:(c + 1) * NC].reshape(k * NC)
                  for c in range(S)]

    def padpos(W, rows):
        return jnp.pad(W.astype(jnp.float32), ((0, rows - 3), (0, 0)))
    bz2, br2, bq2 = (b.astype(jnp.float32).reshape(1, CIN)
                     for b in (bz_pos, br_pos, bq_pos))
    bzl2, brl2, bql2 = (b.astype(jnp.float32).reshape(1, H)
                        for b in (bz_lin, br_lin, bq_lin))
    full = lambda shape: pl.BlockSpec(shape, lambda i: (0, 0))

    def zr_body(nbr_ref, ctr_ref,
                wzp_ref, bz_ref, wzl_ref, bzl_ref,
                wrp_ref, br_ref, wrl_ref, brl_ref,
                z_ref, tab2_ref, rel_ref):
        nb = nbr_ref[...]
        ctr = ctr_ref[...]
        hc, xc_feat = _unpack_bf16_pair(ctr[:, :H])
        xc = ctr[:, H:H + 16]
        feats = []
        for j in range(k):
            fh, fl = _unpack_bf16_pair(nb[j, :, :H])
            feats.append(jnp.concatenate([fh, fl], axis=1))
        relf = jnp.concatenate([nb[j, :, H:H + 16] - xc for j in range(k)],
                               axis=0)
        wz = jnp.tanh(jnp.dot(relf, wzp_ref[...],
                              preferred_element_type=jnp.float32) + bz_ref[...])
        aggz = sum(wz[j * BN:(j + 1) * BN] * feats[j] for j in range(k))
        zz = jax.nn.sigmoid(jnp.dot(aggz, wzl_ref[...],
                                    preferred_element_type=jnp.float32) + bzl_ref[...])
        wr = jnp.tanh(jnp.dot(relf, wrp_ref[...],
                              preferred_element_type=jnp.float32) + br_ref[...])
        aggr = sum(wr[j * BN:(j + 1) * BN] * feats[j] for j in range(k))
        rr = jax.nn.sigmoid(jnp.dot(aggr, wrl_ref[...],
                                    preferred_element_type=jnp.float32) + brl_ref[...])
        z_ref[...] = zz
        tab2_ref[...] = jnp.concatenate([rr * hc, xc_feat], axis=1)
        rel_ref[...] = relf[:, :8].reshape(k, BN, 8)

    def phase2(nbr1_c, ctr_c):
        return pl.pallas_call(
            zr_body,
            grid=(grid,),
            in_specs=[
                pl.BlockSpec((k, BN, 128), lambda i: (0, i, 0)),
                pl.BlockSpec((BN, 128), lambda i: (i, 0)),
                full((16, CIN)), full((1, CIN)), full((CIN, H)), full((1, H)),
                full((16, CIN)), full((1, CIN)), full((CIN, H)), full((1, H)),
            ],
            out_specs=[
                pl.BlockSpec((BN, H), lambda i: (i, 0)),
                pl.BlockSpec((BN, CIN), lambda i: (i, 0)),
                pl.BlockSpec((k, BN, 8), lambda i: (0, i, 0)),
            ],
            out_shape=[
                jax.ShapeDtypeStruct((NC, H), jnp.float32),
                jax.ShapeDtypeStruct((NC, CIN), jnp.float32),
                jax.ShapeDtypeStruct((k, NC, 8), jnp.float32),
            ],
        )(nbr1_c, ctr_c,
          padpos(Wz_pos, 16), bz2, Wz_lin.astype(jnp.float32), bzl2,
          padpos(Wr_pos, 16), br2, Wr_lin.astype(jnp.float32), brl2)

    def q_body(nbr_ref, rel_ref, z_ref, ctr_ref,
               wqp_ref, bq_ref, wql_ref, bql_ref, out_ref):
        nb = nbr_ref[...]
        hc, _ = _unpack_bf16_pair(ctr_ref[...][:, :H])
        wq = jnp.tanh(jnp.dot(rel_ref[...].reshape(k * BN, 8), wqp_ref[...],
                              preferred_element_type=jnp.float32) + bq_ref[...])
        aggq = sum(wq[j * BN:(j + 1) * BN] * nb[j] for j in range(k))
        qq = jnp.tanh(jnp.dot(aggq, wql_ref[...],
                              preferred_element_type=jnp.float32) + bql_ref[...])
        zz = z_ref[...]
        hn = (1.0 - zz) * hc + zz * qq
        eye = jnp.eye(H, dtype=jnp.float32)
        out_ref[...] = lax.dot_general(eye, hn, (((1,), (1,)), ((), ())),
                                       preferred_element_type=jnp.float32)

    def phase4(nbr2_c, rel8_c, z_c, ctr_c):
        return pl.pallas_call(
            q_body,
            grid=(grid,),
            in_specs=[
                pl.BlockSpec((k, BN, CIN), lambda i: (0, i, 0)),
                pl.BlockSpec((k, BN, 8), lambda i: (0, i, 0)),
                pl.BlockSpec((BN, H), lambda i: (i, 0)),
                pl.BlockSpec((BN, 128), lambda i: (i, 0)),
                full((8, CIN)), full((1, CIN)), full((CIN, H)), full((1, H)),
            ],
            out_specs=pl.BlockSpec((H, BN), lambda i: (0, i)),
            out_shape=jax.ShapeDtypeStruct((H, NC), jnp.float32),
        )(nbr2_c, rel8_c, z_c, ctr_c,
          padpos(Wq_pos, 8), bq2, Wq_lin.astype(jnp.float32), bql2)

    def csl(a, c):
        return a[c * NC:(c + 1) * NC]

    nbr1 = [_sc_gather(tab1, idx_chunks[c]).reshape(k, NC, 128)
            for c in range(S)]
    p2 = [phase2(nbr1[c], csl(tab1, c)) for c in range(S)]
    tab2 = jnp.concatenate([p[1] for p in p2], axis=0)

    nbr2 = [_sc_gather(tab2, idx_chunks[c]).reshape(k, NC, CIN)
            for c in range(S)]
    outs = [phase4(nbr2[c], p2[c][2], p2[c][0], csl(tab1, c))
            for c in range(S)]

    h_new = jnp.concatenate(outs, axis=1)
    return h_new[:, :N][None]


# speedup vs baseline: 1.0394x; 1.0098x over previous
import functools

import jax
import jax.numpy as jnp
from jax import lax
from jax.experimental import pallas as pl
from jax.experimental.pallas import tpu as pltpu
from jax.experimental.pallas import tpu_sc as plsc

NW = 32
CHUNK = 448
BN = 3584
PW = 2048
S = 2
NPAD = 100352


def _sc_gather(table, idx):
    total = idx.shape[0]
    per_w = total // NW
    iters = per_w // CHUNK
    assert iters % 2 == 0 and iters * CHUNK == per_w
    nc = 2
    dt = table.dtype

    mesh = plsc.VectorSubcoreMesh(core_axis_name="c", subcore_axis_name="s")

    @functools.partial(
        pl.kernel, mesh=mesh,
        out_type=jax.ShapeDtypeStruct((total, 128), dt),
        scratch_types=[
            pltpu.VMEM((per_w,), jnp.int32),
            pltpu.VMEM((CHUNK, 128), dt),
            pltpu.VMEM((CHUNK, 128), dt),
            pltpu.SemaphoreType.DMA,
            pltpu.SemaphoreType.DMA,
        ],
    )
    def gather_k(idx_hbm, tab_hbm, out_hbm, idx_all, buf0, buf1, sem0, sem1):
        wid = lax.axis_index("s") * nc + lax.axis_index("c")
        base = wid * per_w
        pltpu.sync_copy(idx_hbm.at[pl.ds(base, per_w)], idx_all)

        bufs = (buf0, buf1)
        sems = (sem0, sem1)

        def gcopy(j, b):
            return pltpu.make_async_copy(
                tab_hbm.at[idx_all.at[pl.ds(j * CHUNK, CHUNK)]], bufs[b], sems[b])

        def step(j, b):
            @pl.when(j + 1 < iters)
            def _():
                gcopy(j + 1, 1 - b).start()
            gcopy(j, b).wait()
            pltpu.sync_copy(bufs[b], out_hbm.at[pl.ds(base + j * CHUNK, CHUNK)])

        gcopy(0, 0).start()

        def pair(i2, carry):
            step(i2 * 2, 0)
            step(i2 * 2 + 1, 1)
            return carry

        lax.fori_loop(0, iters // 2, pair, 0)

    return gather_k(idx, table)


def _unpack_bf16_pair(words):
    u = lax.bitcast_convert_type(words, jnp.uint32)
    hi = lax.bitcast_convert_type(u & jnp.uint32(0xFFFF0000), jnp.float32)
    lo = lax.bitcast_convert_type(u << 16, jnp.float32)
    return hi, lo


def kernel(xyz, h, x, knn_indices,
           Wz_pos, bz_pos, Wz_lin, bz_lin,
           Wr_pos, br_pos, Wr_lin, br_lin,
           Wq_pos, bq_pos, Wq_lin, bq_lin):
    h = h.astype(jnp.float32)
    x = x.astype(jnp.float32)
    B, N, _ = xyz.shape
    k = knn_indices.shape[2]
    H = h.shape[1]
    CIN = 2 * H
    Npad = NPAD
    NC = Npad // S
    grid = NC // BN

    def pack_body(h_ref, x_ref, xyz_ref, tab_ref):
        hb = lax.bitcast_convert_type(
            h_ref[...].astype(jnp.bfloat16), jnp.uint16).astype(jnp.uint32)
        xb = lax.bitcast_convert_type(
            x_ref[...].astype(jnp.bfloat16), jnp.uint16).astype(jnp.uint32)
        packed = lax.bitcast_convert_type((hb << 16) | xb, jnp.float32)
        packed_t = jnp.swapaxes(packed, 0, 1)
        xyz16 = jnp.pad(xyz_ref[...], ((0, 0), (0, 13)))
        zer = jnp.zeros((PW, 128 - H - 16), jnp.float32)
        tab_ref[...] = jnp.concatenate([packed_t, xyz16, zer], axis=1)

    xyz0 = xyz[0]
    h0 = h[0]
    x0 = x[0]
    tab1 = pl.pallas_call(
        pack_body,
        grid=(Npad // PW,),
        in_specs=[
            pl.BlockSpec((H, PW), lambda i: (0, i)),
            pl.BlockSpec((H, PW), lambda i: (0, i)),
            pl.BlockSpec((PW, 3), lambda i: (i, 0)),
        ],
        out_specs=pl.BlockSpec((PW, 128), lambda i: (i, 0)),
        out_shape=jax.ShapeDtypeStruct((Npad, 128), jnp.float32),
    )(h0, x0, xyz0)

    idx_km = jnp.pad(knn_indices[0].astype(jnp.int32).T,
                     ((0, 0), (0, Npad - N)))
    idx_chunks = [idx_km[:, c * NC:(c + 1) * NC].reshape(k * NC)
                  for c in range(S)]

    def padpos(W, rows):
        return jnp.pad(W.astype(jnp.float32), ((0, rows - 3), (0, 0)))
    bz2, br2, bq2 = (b.astype(jnp.float32).reshape(1, CIN)
                     for b in (bz_pos, br_pos, bq_pos))
    bzl2, brl2, bql2 = (b.astype(jnp.float32).reshape(1, H)
                        for b in (bz_lin, br_lin, bq_lin))
    full = lambda shape: pl.BlockSpec(shape, lambda i: (0, 0))

    def zr_body(nbr_ref, ctr_ref,
                wzp_ref, bz_ref, wzl_ref, bzl_ref,
                wrp_ref, br_ref, wrl_ref, brl_ref,
                z_ref, tab2_ref, rel_ref):
        nb = nbr_ref[...]
        ctr = ctr_ref[...]
        hc, xc_feat = _unpack_bf16_pair(ctr[:, :H])
        xc = ctr[:, H:H + 16]
        feats = []
        for j in range(k):
            fh, fl = _unpack_bf16_pair(nb[j, :, :H])
            feats.append(jnp.concatenate([fh, fl], axis=1))
        relf = jnp.concatenate([nb[j, :, H:H + 16] - xc for j in range(k)],
                               axis=0)
        wz = jnp.tanh(jnp.dot(relf, wzp_ref[...],
                              preferred_element_type=jnp.float32) + bz_ref[...])
        aggz = sum(wz[j * BN:(j + 1) * BN] * feats[j] for j in range(k))
        zz = jax.nn.sigmoid(jnp.dot(aggz, wzl_ref[...],
                                    preferred_element_type=jnp.float32) + bzl_ref[...])
        wr = jnp.tanh(jnp.dot(relf, wrp_ref[...],
                              preferred_element_type=jnp.float32) + br_ref[...])
        aggr = sum(wr[j * BN:(j + 1) * BN] * feats[j] for j in range(k))
        rr = jax.nn.sigmoid(jnp.dot(aggr, wrl_ref[...],
                                    preferred_element_type=jnp.float32) + brl_ref[...])
        z_ref[...] = zz
        tab2_ref[...] = jnp.concatenate([rr * hc, xc_feat], axis=1)
        rel_ref[...] = relf[:, :8].reshape(k, BN, 8)

    def phase2(nbr1_c, ctr_c):
        return pl.pallas_call(
            zr_body,
            grid=(grid,),
            in_specs=[
                pl.BlockSpec((k, BN, 128), lambda i: (0, i, 0)),
                pl.BlockSpec((BN, 128), lambda i: (i, 0)),
                full((16, CIN)), full((1, CIN)), full((CIN, H)), full((1, H)),
                full((16, CIN)), full((1, CIN)), full((CIN, H)), full((1, H)),
            ],
            out_specs=[
                pl.BlockSpec((BN, H), lambda i: (i, 0)),
                pl.BlockSpec((BN, CIN), lambda i: (i, 0)),
                pl.BlockSpec((k, BN, 8), lambda i: (0, i, 0)),
            ],
            out_shape=[
                jax.ShapeDtypeStruct((NC, H), jnp.float32),
                jax.ShapeDtypeStruct((NC, CIN), jnp.float32),
                jax.ShapeDtypeStruct((k, NC, 8), jnp.float32),
            ],
        )(nbr1_c, ctr_c,
          padpos(Wz_pos, 16), bz2, Wz_lin.astype(jnp.float32), bzl2,
          padpos(Wr_pos, 16), br2, Wr_lin.astype(jnp.float32), brl2)

    def q_body(nbr_ref, rel_ref, z_ref, ctr_ref,
               wqp_ref, bq_ref, wql_ref, bql_ref, out_ref):
        nb = nbr_ref[...]
        hc, _ = _unpack_bf16_pair(ctr_ref[...][:, :H])
        wq = jnp.tanh(jnp.dot(rel_ref[...].reshape(k * BN, 8), wqp_ref[...],
                              preferred_element_type=jnp.float32) + bq_ref[...])
        aggq = sum(wq[j * BN:(j + 1) * BN] * nb[j] for j in range(k))
        qq = jnp.tanh(jnp.dot(aggq, wql_ref[...],
                              preferred_element_type=jnp.float32) + bql_ref[...])
        zz = z_ref[...]
        hn = (1.0 - zz) * hc + zz * qq
        eye = jnp.eye(H, dtype=jnp.float32)
        out_ref[...] = lax.dot_general(eye, hn, (((1,), (1,)), ((), ())),
                                       preferred_element_type=jnp.float32)

    def phase4(nbr2_c, rel8_c, z_c, ctr_c):
        return pl.pallas_call(
            q_body,
            grid=(grid,),
            in_specs=[
                pl.BlockSpec((k, BN, CIN), lambda i: (0, i, 0)),
                pl.BlockSpec((k, BN, 8), lambda i: (0, i, 0)),
                pl.BlockSpec((BN, H), lambda i: (i, 0)),
                pl.BlockSpec((BN, 128), lambda i: (i, 0)),
                full((8, CIN)), full((1, CIN)), full((CIN, H)), full((1, H)),
            ],
            out_specs=pl.BlockSpec((H, BN), lambda i: (0, i)),
            out_shape=jax.ShapeDtypeStruct((H, NC), jnp.float32),
        )(nbr2_c, rel8_c, z_c, ctr_c,
          padpos(Wq_pos, 8), bq2, Wq_lin.astype(jnp.float32), bql2)

    def csl(a, c):
        return a[c * NC:(c + 1) * NC]

    nbr1 = [_sc_gather(tab1, idx_chunks[c]).reshape(k, NC, 128)
            for c in range(S)]
    p2 = [phase2(nbr1[c], csl(tab1, c)) for c in range(S)]
    tab2 = jnp.concatenate([p[1] for p in p2], axis=0)

    nbr2 = [_sc_gather(tab2, idx_chunks[c]).reshape(k, NC, CIN)
            for c in range(S)]
    outs = [phase4(nbr2[c], p2[c][2], p2[c][0], csl(tab1, c))
            for c in range(S)]

    h_new = jnp.concatenate(outs, axis=1)
    return h_new[:, :N][None]


# speedup vs baseline: 1.0517x; 1.0119x over previous
import functools

import jax
import jax.numpy as jnp
from jax import lax
from jax.experimental import pallas as pl
from jax.experimental.pallas import tpu as pltpu
from jax.experimental.pallas import tpu_sc as plsc

NW = 32
CHUNK = 448
BN = 3584
PW = 3584
S = 2
NPAD = 100352


def _sc_gather(table, idx):
    total = idx.shape[0]
    per_w = total // NW
    iters = per_w // CHUNK
    assert iters % 2 == 0 and iters * CHUNK == per_w
    nc = 2
    dt = table.dtype

    mesh = plsc.VectorSubcoreMesh(core_axis_name="c", subcore_axis_name="s")

    @functools.partial(
        pl.kernel, mesh=mesh,
        out_type=jax.ShapeDtypeStruct((total, 128), dt),
        scratch_types=[
            pltpu.VMEM((per_w,), jnp.int32),
            pltpu.VMEM((CHUNK, 128), dt),
            pltpu.VMEM((CHUNK, 128), dt),
            pltpu.SemaphoreType.DMA,
            pltpu.SemaphoreType.DMA,
        ],
    )
    def gather_k(idx_hbm, tab_hbm, out_hbm, idx_all, buf0, buf1, sem0, sem1):
        wid = lax.axis_index("s") * nc + lax.axis_index("c")
        base = wid * per_w
        pltpu.sync_copy(idx_hbm.at[pl.ds(base, per_w)], idx_all)

        bufs = (buf0, buf1)
        sems = (sem0, sem1)

        def gcopy(j, b):
            return pltpu.make_async_copy(
                tab_hbm.at[idx_all.at[pl.ds(j * CHUNK, CHUNK)]], bufs[b], sems[b])

        def step(j, b):
            @pl.when(j + 1 < iters)
            def _():
                gcopy(j + 1, 1 - b).start()
            gcopy(j, b).wait()
            pltpu.sync_copy(bufs[b], out_hbm.at[pl.ds(base + j * CHUNK, CHUNK)])

        gcopy(0, 0).start()

        def pair(i2, carry):
            step(i2 * 2, 0)
            step(i2 * 2 + 1, 1)
            return carry

        lax.fori_loop(0, iters // 2, pair, 0)

    return gather_k(idx, table)


def _unpack_bf16_pair(words):
    u = lax.bitcast_convert_type(words, jnp.uint32)
    hi = lax.bitcast_convert_type(u & jnp.uint32(0xFFFF0000), jnp.float32)
    lo = lax.bitcast_convert_type(u << 16, jnp.float32)
    return hi, lo


def kernel(xyz, h, x, knn_indices,
           Wz_pos, bz_pos, Wz_lin, bz_lin,
           Wr_pos, br_pos, Wr_lin, br_lin,
           Wq_pos, bq_pos, Wq_lin, bq_lin):
    h = h.astype(jnp.float32)
    x = x.astype(jnp.float32)
    B, N, _ = xyz.shape
    k = knn_indices.shape[2]
    H = h.shape[1]
    CIN = 2 * H
    Npad = NPAD
    NC = Npad // S
    grid = NC // BN

    def pack_body(h_ref, x_ref, xyz_ref, tab_ref):
        hb = lax.bitcast_convert_type(
            h_ref[...].astype(jnp.bfloat16), jnp.uint16).astype(jnp.uint32)
        xb = lax.bitcast_convert_type(
            x_ref[...].astype(jnp.bfloat16), jnp.uint16).astype(jnp.uint32)
        packed = lax.bitcast_convert_type((hb << 16) | xb, jnp.float32)
        packed_t = jnp.swapaxes(packed, 0, 1)
        xyz16 = jnp.pad(xyz_ref[...], ((0, 0), (0, 13)))
        zer = jnp.zeros((PW, 128 - H - 16), jnp.float32)
        tab_ref[...] = jnp.concatenate([packed_t, xyz16, zer], axis=1)

    xyz0 = xyz[0]
    h0 = h[0]
    x0 = x[0]
    tab1 = pl.pallas_call(
        pack_body,
        grid=(Npad // PW,),
        in_specs=[
            pl.BlockSpec((H, PW), lambda i: (0, i)),
            pl.BlockSpec((H, PW), lambda i: (0, i)),
            pl.BlockSpec((PW, 3), lambda i: (i, 0)),
        ],
        out_specs=pl.BlockSpec((PW, 128), lambda i: (i, 0)),
        out_shape=jax.ShapeDtypeStruct((Npad, 128), jnp.float32),
    )(h0, x0, xyz0)

    idx_km = jnp.pad(knn_indices[0].astype(jnp.int32).T,
                     ((0, 0), (0, Npad - N)))
    idx_chunks = [idx_km[:, c * NC:(c + 1) * NC].reshape(k * NC)
                  for c in range(S)]

    def padpos(W, rows):
        return jnp.pad(W.astype(jnp.float32), ((0, rows - 3), (0, 0)))
    bz2, br2, bq2 = (b.astype(jnp.float32).reshape(1, CIN)
                     for b in (bz_pos, br_pos, bq_pos))
    bzl2, brl2, bql2 = (b.astype(jnp.float32).reshape(1, H)
                        for b in (bz_lin, br_lin, bq_lin))
    full = lambda shape: pl.BlockSpec(shape, lambda i: (0, 0))

    def zr_body(nbr_ref, ctr_ref,
                wzp_ref, bz_ref, wzl_ref, bzl_ref,
                wrp_ref, br_ref, wrl_ref, brl_ref,
                z_ref, tab2_ref, rel_ref):
        nb = nbr_ref[...]
        ctr = ctr_ref[...]
        hc, xc_feat = _unpack_bf16_pair(ctr[:, :H])
        xc = ctr[:, H:H + 16]
        feats = []
        for j in range(k):
            fh, fl = _unpack_bf16_pair(nb[j, :, :H])
            feats.append(jnp.concatenate([fh, fl], axis=1))
        relf = jnp.concatenate([nb[j, :, H:H + 16] - xc for j in range(k)],
                               axis=0)
        wz = jnp.tanh(jnp.dot(relf, wzp_ref[...],
                              preferred_element_type=jnp.float32) + bz_ref[...])
        aggz = sum(wz[j * BN:(j + 1) * BN] * feats[j] for j in range(k))
        zz = jax.nn.sigmoid(jnp.dot(aggz, wzl_ref[...],
                                    preferred_element_type=jnp.float32) + bzl_ref[...])
        wr = jnp.tanh(jnp.dot(relf, wrp_ref[...],
                              preferred_element_type=jnp.float32) + br_ref[...])
        aggr = sum(wr[j * BN:(j + 1) * BN] * feats[j] for j in range(k))
        rr = jax.nn.sigmoid(jnp.dot(aggr, wrl_ref[...],
                                    preferred_element_type=jnp.float32) + brl_ref[...])
        z_ref[...] = zz
        tab2_ref[...] = jnp.concatenate([rr * hc, xc_feat], axis=1)
        rel_ref[...] = relf[:, :8].reshape(k, BN, 8)

    def phase2(nbr1_c, ctr_c):
        return pl.pallas_call(
            zr_body,
            grid=(grid,),
            in_specs=[
                pl.BlockSpec((k, BN, 128), lambda i: (0, i, 0)),
                pl.BlockSpec((BN, 128), lambda i: (i, 0)),
                full((16, CIN)), full((1, CIN)), full((CIN, H)), full((1, H)),
                full((16, CIN)), full((1, CIN)), full((CIN, H)), full((1, H)),
            ],
            out_specs=[
                pl.BlockSpec((BN, H), lambda i: (i, 0)),
                pl.BlockSpec((BN, CIN), lambda i: (i, 0)),
                pl.BlockSpec((k, BN, 8), lambda i: (0, i, 0)),
            ],
            out_shape=[
                jax.ShapeDtypeStruct((NC, H), jnp.float32),
                jax.ShapeDtypeStruct((NC, CIN), jnp.float32),
                jax.ShapeDtypeStruct((k, NC, 8), jnp.float32),
            ],
        )(nbr1_c, ctr_c,
          padpos(Wz_pos, 16), bz2, Wz_lin.astype(jnp.float32), bzl2,
          padpos(Wr_pos, 16), br2, Wr_lin.astype(jnp.float32), brl2)

    def q_body(nbr_ref, rel_ref, z_ref, ctr_ref,
               wqp_ref, bq_ref, wql_ref, bql_ref, out_ref):
        nb = nbr_ref[...]
        hc, _ = _unpack_bf16_pair(ctr_ref[...][:, :H])
        wq = jnp.tanh(jnp.dot(rel_ref[...].reshape(k * BN, 8), wqp_ref[...],
                              preferred_element_type=jnp.float32) + bq_ref[...])
        aggq = sum(wq[j * BN:(j + 1) * BN] * nb[j] for j in range(k))
        qq = jnp.tanh(jnp.dot(aggq, wql_ref[...],
                              preferred_element_type=jnp.float32) + bql_ref[...])
        zz = z_ref[...]
        hn = (1.0 - zz) * hc + zz * qq
        eye = jnp.eye(H, dtype=jnp.float32)
        out_ref[...] = lax.dot_general(eye, hn, (((1,), (1,)), ((), ())),
                                       preferred_element_type=jnp.float32)

    def phase4(nbr2_c, rel8_c, z_c, ctr_c):
        return pl.pallas_call(
            q_body,
            grid=(grid,),
            in_specs=[
                pl.BlockSpec((k, BN, CIN), lambda i: (0, i, 0)),
                pl.BlockSpec((k, BN, 8), lambda i: (0, i, 0)),
                pl.BlockSpec((BN, H), lambda i: (i, 0)),
                pl.BlockSpec((BN, 128), lambda i: (i, 0)),
                full((8, CIN)), full((1, CIN)), full((CIN, H)), full((1, H)),
            ],
            out_specs=pl.BlockSpec((H, BN), lambda i: (0, i)),
            out_shape=jax.ShapeDtypeStruct((H, NC), jnp.float32),
        )(nbr2_c, rel8_c, z_c, ctr_c,
          padpos(Wq_pos, 8), bq2, Wq_lin.astype(jnp.float32), bql2)

    def csl(a, c):
        return a[c * NC:(c + 1) * NC]

    nbr1 = [_sc_gather(tab1, idx_chunks[c]).reshape(k, NC, 128)
            for c in range(S)]
    p2 = [phase2(nbr1[c], csl(tab1, c)) for c in range(S)]
    tab2 = jnp.concatenate([p[1] for p in p2], axis=0)

    nbr2 = [_sc_gather(tab2, idx_chunks[c]).reshape(k, NC, CIN)
            for c in range(S)]
    outs = [phase4(nbr2[c], p2[c][2], p2[c][0], csl(tab1, c))
            for c in range(S)]

    h_new = jnp.concatenate(outs, axis=1)
    return h_new[:, :N][None]
